# baseline (speedup 1.0000x reference)
import functools

import jax
import jax.numpy as jnp
from jax import lax
from jax.experimental import pallas as pl
from jax.experimental.pallas import tpu as pltpu
from jax.experimental.pallas import tpu_sc as plsc

BATCH = 16384
EMBED_DIM = 64
NUMERIC_DIM = 16
OUT_DIM = 2 * EMBED_DIM + NUMERIC_DIM

_info = plsc.get_sparse_core_info()
_NC, _NS = _info.num_cores, _info.num_subcores
_NW = _NC * _NS
_BPW = BATCH // _NW


@functools.partial(
    pl.kernel,
    mesh=plsc.VectorSubcoreMesh(core_axis_name="c", subcore_axis_name="s"),
    out_type=jax.ShapeDtypeStruct((BATCH, OUT_DIM), jnp.float32),
    scratch_types=[
        pltpu.VMEM((_BPW,), jnp.int32),
        pltpu.VMEM((_BPW,), jnp.int32),
        pltpu.VMEM((_BPW, EMBED_DIM), jnp.float32),
        pltpu.VMEM((_BPW, EMBED_DIM), jnp.float32),
        pltpu.VMEM((_BPW, NUMERIC_DIM), jnp.float32),
        pltpu.SemaphoreType.DMA,
    ],
    compiler_params=pltpu.CompilerParams(use_tc_tiling_on_sc=False),
)
def _encode(sex_hbm, site_hbm, numeric_hbm, sex_table_hbm, site_table_hbm,
            out_hbm, sex_idx, site_idx, sex_rows, site_rows, num_v, sem):
    wid = lax.axis_index("s") * _NC + lax.axis_index("c")
    base = wid * _BPW
    pltpu.sync_copy(sex_hbm.at[pl.ds(base, _BPW)], sex_idx)
    pltpu.sync_copy(site_hbm.at[pl.ds(base, _BPW)], site_idx)
    g_sex = pltpu.async_copy(sex_table_hbm.at[sex_idx], sex_rows, sem)
    g_site = pltpu.async_copy(site_table_hbm.at[site_idx], site_rows, sem)
    g_num = pltpu.async_copy(numeric_hbm.at[pl.ds(base, _BPW)], num_v, sem)
    g_sex.wait()
    g_site.wait()
    g_num.wait()
    pltpu.sync_copy(sex_rows, out_hbm.at[pl.ds(base, _BPW), pl.ds(0, EMBED_DIM)])
    pltpu.sync_copy(site_rows,
                    out_hbm.at[pl.ds(base, _BPW), pl.ds(EMBED_DIM, EMBED_DIM)])
    pltpu.sync_copy(num_v,
                    out_hbm.at[pl.ds(base, _BPW), pl.ds(2 * EMBED_DIM, NUMERIC_DIM)])


def kernel(sex, site, numeric, sex_table, site_table):
    return _encode(sex, site, numeric, sex_table, site_table)
